# baseline (device time: 46002 ns/iter reference)
import jax
import jax.numpy as jnp
from jax import lax
from jax.experimental import pallas as pl
from jax.experimental.pallas import tpu as pltpu

N_DEV = 4


def kernel(A, B):
    m, k = A.shape
    _, n = B.shape

    def body(a_ref, b_ref, out_ref, comm_ref, send_sems, recv_sems):
        my_pos = lax.axis_index("i")
        left = (my_pos - 1) % N_DEV
        right = (my_pos + 1) % N_DEV

        comm_ref[0, :, :] = jnp.dot(
            a_ref[:, :], b_ref[:, :], preferred_element_type=jnp.float32
        )

        barrier_sem = pltpu.get_barrier_semaphore()
        for nbr in (left, right):
            pl.semaphore_signal(
                barrier_sem, inc=1,
                device_id=(nbr,), device_id_type=pl.DeviceIdType.MESH,
            )
        pl.semaphore_wait(barrier_sem, 2)

        acc = comm_ref[0, :, :]
        for h in range(N_DEV - 1):
            rdma = pltpu.make_async_remote_copy(
                src_ref=comm_ref.at[h],
                dst_ref=comm_ref.at[h + 1],
                send_sem=send_sems.at[h],
                recv_sem=recv_sems.at[h],
                device_id=(right,),
                device_id_type=pl.DeviceIdType.MESH,
            )
            rdma.start()
            rdma.wait()
            acc = acc + comm_ref[h + 1, :, :]

        z = acc
        out_ref[:, :] = 0.5 * z * (
            1.0 + jnp.tanh(0.7978845608 * (z + 0.044715 * z * z * z))
        )

    return pl.pallas_call(
        body,
        out_shape=jax.ShapeDtypeStruct((m, n), jnp.float32),
        in_specs=[
            pl.BlockSpec(memory_space=pltpu.VMEM),
            pl.BlockSpec(memory_space=pltpu.VMEM),
        ],
        out_specs=pl.BlockSpec(memory_space=pltpu.VMEM),
        scratch_shapes=[
            pltpu.VMEM((N_DEV, m, n), jnp.float32),
            pltpu.SemaphoreType.DMA((N_DEV - 1,)),
            pltpu.SemaphoreType.DMA((N_DEV - 1,)),
        ],
        compiler_params=pltpu.CompilerParams(collective_id=0),
    )(A, B)


# device time: 21635 ns/iter; 2.1263x vs baseline; 2.1263x over previous
import jax
import jax.numpy as jnp
from jax import lax
from jax.experimental import pallas as pl
from jax.experimental.pallas import tpu as pltpu

N_DEV = 4


def _gelu(z):
    return 0.5 * z * (1.0 + jnp.tanh(0.7978845608 * (z + 0.044715 * z * z * z)))


def kernel(A, B):
    m, k = A.shape
    _, n = B.shape
    half = m // 2
    h2 = half // 2
    h4 = half // 4

    def body(a_ref, b_ref, out_ref, z_ref, rs1_ref, rs2_ref,
             send_sems, recv_sems):
        my = lax.axis_index("i")
        cx = my // 2
        ry = my % 2
        cy = ry + cx - 2 * ry * cx
        py = my + 1 - 2 * ry
        px = 3 - my

        z_ref[:, :] = jnp.dot(
            a_ref[:, :], b_ref[:, :], preferred_element_type=jnp.float32
        )

        barrier_sem = pltpu.get_barrier_semaphore()
        for nbr in (py, px):
            pl.semaphore_signal(
                barrier_sem, inc=1,
                device_id=(nbr,), device_id_type=pl.DeviceIdType.MESH,
            )
        pl.semaphore_wait(barrier_sem, 2)

        cfg = [(0, cy, py, cx, px), (half, cx, px, cy, py)]

        def exchange(stage, descrs):
            started = []
            for (src, dst, sem_idx, dev) in descrs:
                r = pltpu.make_async_remote_copy(
                    src_ref=src, dst_ref=dst,
                    send_sem=send_sems.at[sem_idx],
                    recv_sem=recv_sems.at[sem_idx],
                    device_id=(dev,), device_id_type=pl.DeviceIdType.MESH,
                )
                r.start()
                started.append(r)
            for r in started:
                r.wait()

        exchange(0, [
            (z_ref.at[pl.ds(base + h2 * (1 - s1), h2), :], rs1_ref.at[b],
             (b, 0), P1)
            for b, (base, s1, P1, s2, P2) in enumerate(cfg)
        ])
        for b, (base, s1, P1, s2, P2) in enumerate(cfg):
            o1 = base + h2 * s1
            z_ref[pl.ds(o1, h2), :] = z_ref[pl.ds(o1, h2), :] + rs1_ref[b]

        exchange(1, [
            (z_ref.at[pl.ds(base + h2 * s1 + h4 * (1 - s2), h4), :],
             rs2_ref.at[b], (b, 1), P2)
            for b, (base, s1, P1, s2, P2) in enumerate(cfg)
        ])
        for b, (base, s1, P1, s2, P2) in enumerate(cfg):
            o2 = base + h2 * s1 + h4 * s2
            zq = z_ref[pl.ds(o2, h4), :] + rs2_ref[b]
            out_ref[pl.ds(o2, h4), :] = _gelu(zq)

        exchange(2, [
            (out_ref.at[pl.ds(base + h2 * s1 + h4 * s2, h4), :],
             out_ref.at[pl.ds(base + h2 * s1 + h4 * s2, h4), :],
             (b, 2), P2)
            for b, (base, s1, P1, s2, P2) in enumerate(cfg)
        ])

        exchange(3, [
            (out_ref.at[pl.ds(base + h2 * s1, h2), :],
             out_ref.at[pl.ds(base + h2 * s1, h2), :],
             (b, 3), P1)
            for b, (base, s1, P1, s2, P2) in enumerate(cfg)
        ])

    return pl.pallas_call(
        body,
        out_shape=jax.ShapeDtypeStruct((m, n), jnp.float32),
        in_specs=[
            pl.BlockSpec(memory_space=pltpu.VMEM),
            pl.BlockSpec(memory_space=pltpu.VMEM),
        ],
        out_specs=pl.BlockSpec(memory_space=pltpu.VMEM),
        scratch_shapes=[
            pltpu.VMEM((m, n), jnp.float32),
            pltpu.VMEM((2, h2, n), jnp.float32),
            pltpu.VMEM((2, h4, n), jnp.float32),
            pltpu.SemaphoreType.DMA((2, 4)),
            pltpu.SemaphoreType.DMA((2, 4)),
        ],
        compiler_params=pltpu.CompilerParams(collective_id=0),
    )(A, B)


# device time: 20072 ns/iter; 2.2918x vs baseline; 1.0779x over previous
import jax
import jax.numpy as jnp
from jax import lax
from jax.experimental import pallas as pl
from jax.experimental.pallas import tpu as pltpu

N_DEV = 4


def _gelu(z):
    return 0.5 * z * (1.0 + jnp.tanh(0.7978845608 * (z + 0.044715 * z * z * z)))


def kernel(A, B):
    m, k = A.shape
    _, n = B.shape
    half = m // 2
    h2 = half // 2

    def body(a_ref, b_ref, out_ref, z_ref, rs1_ref, ps_ref,
             send_sems, recv_sems):
        my = lax.axis_index("i")
        cx = my // 2
        ry = my % 2
        cy = ry + cx - 2 * ry * cx
        py = my + 1 - 2 * ry
        px = 3 - my

        barrier_sem = pltpu.get_barrier_semaphore()
        for nbr in (py, px):
            pl.semaphore_signal(
                barrier_sem, inc=1,
                device_id=(nbr,), device_id_type=pl.DeviceIdType.MESH,
            )

        cfg = [(0, cy, py, px), (half, cx, px, py)]

        def start_rdma(src, dst, sem_idx, dev):
            r = pltpu.make_async_remote_copy(
                src_ref=src, dst_ref=dst,
                send_sem=send_sems.at[sem_idx],
                recv_sem=recv_sems.at[sem_idx],
                device_id=(dev,), device_id_type=pl.DeviceIdType.MESH,
            )
            r.start()
            return r

        for base, s1, P1, P2 in cfg:
            snd = base + h2 * (1 - s1)
            z_ref[pl.ds(snd, h2), :] = jnp.dot(
                a_ref[pl.ds(snd, h2), :], b_ref[:, :],
                preferred_element_type=jnp.float32,
            )
        pl.semaphore_wait(barrier_sem, 2)
        rs1 = [
            start_rdma(z_ref.at[pl.ds(base + h2 * (1 - s1), h2), :],
                       rs1_ref.at[b], (b, 0), P1)
            for b, (base, s1, P1, P2) in enumerate(cfg)
        ]
        for base, s1, P1, P2 in cfg:
            o1 = base + h2 * s1
            z_ref[pl.ds(o1, h2), :] = jnp.dot(
                a_ref[pl.ds(o1, h2), :], b_ref[:, :],
                preferred_element_type=jnp.float32,
            )
        for r in rs1:
            r.wait_recv()

        for b, (base, s1, P1, P2) in enumerate(cfg):
            o1 = base + h2 * s1
            z_ref[pl.ds(o1, h2), :] = z_ref[pl.ds(o1, h2), :] + rs1_ref[b]

        ps = [
            start_rdma(z_ref.at[pl.ds(base + h2 * s1, h2), :],
                       ps_ref.at[b], (b, 1), P2)
            for b, (base, s1, P1, P2) in enumerate(cfg)
        ]
        for r in ps:
            r.wait_recv()

        for b, (base, s1, P1, P2) in enumerate(cfg):
            o1 = base + h2 * s1
            out_ref[pl.ds(o1, h2), :] = _gelu(
                z_ref[pl.ds(o1, h2), :] + ps_ref[b]
            )

        ag = [
            start_rdma(out_ref.at[pl.ds(base + h2 * s1, h2), :],
                       out_ref.at[pl.ds(base + h2 * s1, h2), :],
                       (b, 2), P1)
            for b, (base, s1, P1, P2) in enumerate(cfg)
        ]
        for r in ag:
            r.wait_recv()

        for r in rs1 + ps + ag:
            r.wait_send()

    return pl.pallas_call(
        body,
        out_shape=jax.ShapeDtypeStruct((m, n), jnp.float32),
        in_specs=[
            pl.BlockSpec(memory_space=pltpu.VMEM),
            pl.BlockSpec(memory_space=pltpu.VMEM),
        ],
        out_specs=pl.BlockSpec(memory_space=pltpu.VMEM),
        scratch_shapes=[
            pltpu.VMEM((m, n), jnp.float32),
            pltpu.VMEM((2, h2, n), jnp.float32),
            pltpu.VMEM((2, h2, n), jnp.float32),
            pltpu.SemaphoreType.DMA((2, 3)),
            pltpu.SemaphoreType.DMA((2, 3)),
        ],
        compiler_params=pltpu.CompilerParams(collective_id=0),
    )(A, B)


# device time: 18641 ns/iter; 2.4678x vs baseline; 1.0768x over previous
import jax
import jax.numpy as jnp
from jax import lax
from jax.experimental import pallas as pl
from jax.experimental.pallas import tpu as pltpu

N_DEV = 4


def _gelu(z):
    return 0.5 * z * (1.0 + jnp.tanh(0.7978845608 * (z + 0.044715 * z * z * z)))


def kernel(A, B):
    m, k = A.shape
    _, n = B.shape
    half = m // 2
    h2 = half // 2

    def body(a_ref, b_ref, out_ref, z_ref, rs1_ref, ps_ref,
             send_sems, recv_sems):
        my = lax.axis_index("i")
        cx = my // 2
        ry = my % 2
        cy = ry + cx - 2 * ry * cx
        py = my + 1 - 2 * ry
        px = 3 - my

        barrier_sem = pltpu.get_barrier_semaphore()
        for nbr in (py, px):
            pl.semaphore_signal(
                barrier_sem, inc=1,
                device_id=(nbr,), device_id_type=pl.DeviceIdType.MESH,
            )

        cfg = [(0, cy, py, px), (half, cx, px, py)]

        def start_rdma(src, dst, sem_idx, dev):
            r = pltpu.make_async_remote_copy(
                src_ref=src, dst_ref=dst,
                send_sem=send_sems.at[sem_idx],
                recv_sem=recv_sems.at[sem_idx],
                device_id=(dev,), device_id_type=pl.DeviceIdType.MESH,
            )
            r.start()
            return r

        for base, s1, P1, P2 in cfg:
            snd = base + h2 * (1 - s1)
            z_ref[pl.ds(snd, h2), :] = jnp.dot(
                a_ref[pl.ds(snd, h2), :], b_ref[:, :],
                preferred_element_type=jnp.float32,
            )
        pl.semaphore_wait(barrier_sem, 2)
        rs1 = [
            start_rdma(z_ref.at[pl.ds(base + h2 * (1 - s1), h2), :],
                       rs1_ref.at[b], (b, 0), P1)
            for b, (base, s1, P1, P2) in enumerate(cfg)
        ]
        for base, s1, P1, P2 in cfg:
            o1 = base + h2 * s1
            z_ref[pl.ds(o1, h2), :] = jnp.dot(
                a_ref[pl.ds(o1, h2), :], b_ref[:, :],
                preferred_element_type=jnp.float32,
            )
        for r in rs1:
            r.wait_recv()

        cs = n // 2
        ps = []
        for c in (0, 1):
            for b, (base, s1, P1, P2) in enumerate(cfg):
                o1 = base + h2 * s1
                z_ref[pl.ds(o1, h2), pl.ds(c * cs, cs)] = (
                    z_ref[pl.ds(o1, h2), pl.ds(c * cs, cs)]
                    + rs1_ref[b, :, pl.ds(c * cs, cs)]
                )
            ps.append([
                start_rdma(z_ref.at[pl.ds(base + h2 * s1, h2), pl.ds(c * cs, cs)],
                           ps_ref.at[b, :, pl.ds(c * cs, cs)],
                           (b, 1 + c), P2)
                for b, (base, s1, P1, P2) in enumerate(cfg)
            ])

        ag = []
        for c in (0, 1):
            for r in ps[c]:
                r.wait_recv()
            for b, (base, s1, P1, P2) in enumerate(cfg):
                o1 = base + h2 * s1
                out_ref[pl.ds(o1, h2), pl.ds(c * cs, cs)] = _gelu(
                    z_ref[pl.ds(o1, h2), pl.ds(c * cs, cs)]
                    + ps_ref[b, :, pl.ds(c * cs, cs)]
                )
            ag.append([
                start_rdma(out_ref.at[pl.ds(base + h2 * s1, h2), pl.ds(c * cs, cs)],
                           out_ref.at[pl.ds(base + h2 * s1, h2), pl.ds(c * cs, cs)],
                           (b, 3 + c), P1)
                for b, (base, s1, P1, P2) in enumerate(cfg)
            ])
        for grp in ag:
            for r in grp:
                r.wait_recv()

        for r in rs1 + ps[0] + ps[1] + ag[0] + ag[1]:
            r.wait_send()

    return pl.pallas_call(
        body,
        out_shape=jax.ShapeDtypeStruct((m, n), jnp.float32),
        in_specs=[
            pl.BlockSpec(memory_space=pltpu.VMEM),
            pl.BlockSpec(memory_space=pltpu.VMEM),
        ],
        out_specs=pl.BlockSpec(memory_space=pltpu.VMEM),
        scratch_shapes=[
            pltpu.VMEM((m, n), jnp.float32),
            pltpu.VMEM((2, h2, n), jnp.float32),
            pltpu.VMEM((2, h2, n), jnp.float32),
            pltpu.SemaphoreType.DMA((2, 5)),
            pltpu.SemaphoreType.DMA((2, 5)),
        ],
        compiler_params=pltpu.CompilerParams(collective_id=0),
    )(A, B)


# device time: 17288 ns/iter; 2.6609x vs baseline; 1.0783x over previous
import jax
import jax.numpy as jnp
from jax import lax
from jax.experimental import pallas as pl
from jax.experimental.pallas import tpu as pltpu

N_DEV = 4


def _gelu(z):
    return 0.5 * z * (1.0 + jnp.tanh(0.7978845608 * (z + 0.044715 * z * z * z)))


def kernel(A, B):
    m, k = A.shape
    _, n = B.shape
    half = m // 2
    h2 = half // 2

    def body(a_ref, b_ref, out_ref, z_ref, rs1_ref, ps_ref,
             send_sems, recv_sems):
        my = lax.axis_index("i")
        cx = my // 2
        ry = my % 2
        cy = ry + cx - 2 * ry * cx
        py = my + 1 - 2 * ry
        px = 3 - my

        barrier_sem = pltpu.get_barrier_semaphore()
        for nbr in (py, px):
            pl.semaphore_signal(
                barrier_sem, inc=1,
                device_id=(nbr,), device_id_type=pl.DeviceIdType.MESH,
            )

        cfg = [(0, cy, py, px), (half, cx, px, py)]

        def start_rdma(src, dst, sem_idx, dev):
            r = pltpu.make_async_remote_copy(
                src_ref=src, dst_ref=dst,
                send_sem=send_sems.at[sem_idx],
                recv_sem=recv_sems.at[sem_idx],
                device_id=(dev,), device_id_type=pl.DeviceIdType.MESH,
            )
            r.start()
            return r

        cs = n // 2
        rs1 = []
        for c in (0, 1):
            for base, s1, P1, P2 in cfg:
                snd = base + h2 * (1 - s1)
                z_ref[pl.ds(snd, h2), pl.ds(c * cs, cs)] = jnp.dot(
                    a_ref[pl.ds(snd, h2), :], b_ref[:, pl.ds(c * cs, cs)],
                    preferred_element_type=jnp.float32,
                )
            if c == 0:
                pl.semaphore_wait(barrier_sem, 2)
            rs1.append([
                start_rdma(
                    z_ref.at[pl.ds(base + h2 * (1 - s1), h2), pl.ds(c * cs, cs)],
                    rs1_ref.at[b, :, pl.ds(c * cs, cs)],
                    (b, 0 + c), P1)
                for b, (base, s1, P1, P2) in enumerate(cfg)
            ])
        for base, s1, P1, P2 in cfg:
            o1 = base + h2 * s1
            z_ref[pl.ds(o1, h2), :] = jnp.dot(
                a_ref[pl.ds(o1, h2), :], b_ref[:, :],
                preferred_element_type=jnp.float32,
            )

        ps = []
        for c in (0, 1):
            for r in rs1[c]:
                r.wait_recv()
            for b, (base, s1, P1, P2) in enumerate(cfg):
                o1 = base + h2 * s1
                z_ref[pl.ds(o1, h2), pl.ds(c * cs, cs)] = (
                    z_ref[pl.ds(o1, h2), pl.ds(c * cs, cs)]
                    + rs1_ref[b, :, pl.ds(c * cs, cs)]
                )
            ps.append([
                start_rdma(z_ref.at[pl.ds(base + h2 * s1, h2), pl.ds(c * cs, cs)],
                           ps_ref.at[b, :, pl.ds(c * cs, cs)],
                           (b, 2 + c), P2)
                for b, (base, s1, P1, P2) in enumerate(cfg)
            ])

        ag = []
        for c in (0, 1):
            for r in ps[c]:
                r.wait_recv()
            for b, (base, s1, P1, P2) in enumerate(cfg):
                o1 = base + h2 * s1
                out_ref[pl.ds(o1, h2), pl.ds(c * cs, cs)] = _gelu(
                    z_ref[pl.ds(o1, h2), pl.ds(c * cs, cs)]
                    + ps_ref[b, :, pl.ds(c * cs, cs)]
                )
            ag.append([
                start_rdma(out_ref.at[pl.ds(base + h2 * s1, h2), pl.ds(c * cs, cs)],
                           out_ref.at[pl.ds(base + h2 * s1, h2), pl.ds(c * cs, cs)],
                           (b, 4 + c), P1)
                for b, (base, s1, P1, P2) in enumerate(cfg)
            ])
        for grp in ag:
            for r in grp:
                r.wait_recv()

        for r in rs1[0] + rs1[1] + ps[0] + ps[1] + ag[0] + ag[1]:
            r.wait_send()

    return pl.pallas_call(
        body,
        out_shape=jax.ShapeDtypeStruct((m, n), jnp.float32),
        in_specs=[
            pl.BlockSpec(memory_space=pltpu.VMEM),
            pl.BlockSpec(memory_space=pltpu.VMEM),
        ],
        out_specs=pl.BlockSpec(memory_space=pltpu.VMEM),
        scratch_shapes=[
            pltpu.VMEM((m, n), jnp.float32),
            pltpu.VMEM((2, h2, n), jnp.float32),
            pltpu.VMEM((2, h2, n), jnp.float32),
            pltpu.SemaphoreType.DMA((2, 6)),
            pltpu.SemaphoreType.DMA((2, 6)),
        ],
        compiler_params=pltpu.CompilerParams(collective_id=0),
    )(A, B)


# device time: 16794 ns/iter; 2.7392x vs baseline; 1.0294x over previous
import jax
import jax.numpy as jnp
from jax import lax
from jax.experimental import pallas as pl
from jax.experimental.pallas import tpu as pltpu

N_DEV = 4
CHUNKS = 4


def _gelu(z):
    return 0.5 * z * (1.0 + jnp.tanh(0.7978845608 * (z + 0.044715 * z * z * z)))


def kernel(A, B):
    m, k = A.shape
    _, n = B.shape
    half = m // 2
    h2 = half // 2

    def body(a_ref, b_ref, out_ref, z_ref, rs1_ref, ps_ref,
             send_sems, recv_sems):
        my = lax.axis_index("i")
        cx = my // 2
        ry = my % 2
        cy = ry + cx - 2 * ry * cx
        py = my + 1 - 2 * ry
        px = 3 - my

        barrier_sem = pltpu.get_barrier_semaphore()
        for nbr in (py, px):
            pl.semaphore_signal(
                barrier_sem, inc=1,
                device_id=(nbr,), device_id_type=pl.DeviceIdType.MESH,
            )

        cfg = [(0, cy, py, px), (half, cx, px, py)]

        def start_rdma(src, dst, sem_idx, dev):
            r = pltpu.make_async_remote_copy(
                src_ref=src, dst_ref=dst,
                send_sem=send_sems.at[sem_idx],
                recv_sem=recv_sems.at[sem_idx],
                device_id=(dev,), device_id_type=pl.DeviceIdType.MESH,
            )
            r.start()
            return r

        chunks = range(CHUNKS)
        cs = n // CHUNKS
        rs1 = []
        for c in chunks:
            for base, s1, P1, P2 in cfg:
                snd = base + h2 * (1 - s1)
                z_ref[pl.ds(snd, h2), pl.ds(c * cs, cs)] = jnp.dot(
                    a_ref[pl.ds(snd, h2), :], b_ref[:, pl.ds(c * cs, cs)],
                    preferred_element_type=jnp.float32,
                )
            if c == 0:
                pl.semaphore_wait(barrier_sem, 2)
            rs1.append([
                start_rdma(
                    z_ref.at[pl.ds(base + h2 * (1 - s1), h2), pl.ds(c * cs, cs)],
                    rs1_ref.at[b, :, pl.ds(c * cs, cs)],
                    (b, c), P1)
                for b, (base, s1, P1, P2) in enumerate(cfg)
            ])
        for base, s1, P1, P2 in cfg:
            o1 = base + h2 * s1
            z_ref[pl.ds(o1, h2), :] = jnp.dot(
                a_ref[pl.ds(o1, h2), :], b_ref[:, :],
                preferred_element_type=jnp.float32,
            )

        ps = []
        for c in chunks:
            for r in rs1[c]:
                r.wait_recv()
            for b, (base, s1, P1, P2) in enumerate(cfg):
                o1 = base + h2 * s1
                z_ref[pl.ds(o1, h2), pl.ds(c * cs, cs)] = (
                    z_ref[pl.ds(o1, h2), pl.ds(c * cs, cs)]
                    + rs1_ref[b, :, pl.ds(c * cs, cs)]
                )
            ps.append([
                start_rdma(z_ref.at[pl.ds(base + h2 * s1, h2), pl.ds(c * cs, cs)],
                           ps_ref.at[b, :, pl.ds(c * cs, cs)],
                           (b, CHUNKS + c), P2)
                for b, (base, s1, P1, P2) in enumerate(cfg)
            ])

        ag = []
        for c in chunks:
            for r in ps[c]:
                r.wait_recv()
            for b, (base, s1, P1, P2) in enumerate(cfg):
                o1 = base + h2 * s1
                out_ref[pl.ds(o1, h2), pl.ds(c * cs, cs)] = _gelu(
                    z_ref[pl.ds(o1, h2), pl.ds(c * cs, cs)]
                    + ps_ref[b, :, pl.ds(c * cs, cs)]
                )
            ag.append([
                start_rdma(out_ref.at[pl.ds(base + h2 * s1, h2), pl.ds(c * cs, cs)],
                           out_ref.at[pl.ds(base + h2 * s1, h2), pl.ds(c * cs, cs)],
                           (b, 2 * CHUNKS + c), P1)
                for b, (base, s1, P1, P2) in enumerate(cfg)
            ])
        for grp in ag:
            for r in grp:
                r.wait_recv()

        for grp in rs1 + ps + ag:
            for r in grp:
                r.wait_send()

    return pl.pallas_call(
        body,
        out_shape=jax.ShapeDtypeStruct((m, n), jnp.float32),
        in_specs=[
            pl.BlockSpec(memory_space=pltpu.VMEM),
            pl.BlockSpec(memory_space=pltpu.VMEM),
        ],
        out_specs=pl.BlockSpec(memory_space=pltpu.VMEM),
        scratch_shapes=[
            pltpu.VMEM((m, n), jnp.float32),
            pltpu.VMEM((2, h2, n), jnp.float32),
            pltpu.VMEM((2, h2, n), jnp.float32),
            pltpu.SemaphoreType.DMA((2, 3 * CHUNKS)),
            pltpu.SemaphoreType.DMA((2, 3 * CHUNKS)),
        ],
        compiler_params=pltpu.CompilerParams(collective_id=0),
    )(A, B)
